# baseline (speedup 1.0000x reference)
import functools

import jax
import jax.numpy as jnp
from jax import lax
from jax.experimental import pallas as pl
from jax.experimental.pallas import tpu as pltpu
from jax.experimental.pallas import tpu_sc as plsc

NC = 2
NS = 16
L = 16
NW = NC * NS

_MESH = dict(core_axis_name="c", subcore_axis_name="s",
             num_cores=NC, num_subcores=NS)
_SC_PARAMS = pltpu.CompilerParams(needs_layout_passes=False)


def _wid():
    return lax.axis_index("c") * NS + lax.axis_index("s")


def _make_deg(ET, NP):
    EW = ET // NW
    steps = EW // L

    @functools.partial(
        pl.kernel,
        out_type=jax.ShapeDtypeStruct((NW, NP), jnp.float32),
        mesh=plsc.VectorSubcoreMesh(**_MESH),
        compiler_params=_SC_PARAMS,
        scratch_types=[
            pltpu.VMEM((EW,), jnp.int32),
            pltpu.VMEM((EW,), jnp.float32),
            pltpu.VMEM((NP,), jnp.float32),
            pltpu.SemaphoreType.DMA,
        ],
    )
    def deg_kernel(dst_hbm, w_hbm, out_hbm, dstv, wv, acc, sem):
        w = _wid()
        sl_edges = pl.ds(w * EW, EW)
        cps = [
            pltpu.async_copy(dst_hbm.at[sl_edges], dstv, sem),
            pltpu.async_copy(w_hbm.at[sl_edges], wv, sem),
        ]
        zero = jnp.zeros((L,), jnp.float32)

        @plsc.parallel_loop(0, NP // L, unroll=8)
        def _(i):
            acc[pl.ds(i * L, L)] = zero

        for cp in cps:
            cp.wait()

        @plsc.parallel_loop(0, steps, unroll=4)
        def _(i):
            sl = pl.ds(i * L, L)
            plsc.addupdate_scatter(acc, [dstv[sl]], wv[sl])

        pltpu.async_copy(acc, out_hbm.at[w], sem).wait()

    return deg_kernel


def _make_agg_norm(ET, NP, K):
    EW = ET // NW
    steps = EW // L
    NPK = NP * K

    @functools.partial(
        pl.kernel,
        out_type=(jax.ShapeDtypeStruct((NW, NPK), jnp.float32),
                  jax.ShapeDtypeStruct((ET,), jnp.float32)),
        mesh=plsc.VectorSubcoreMesh(**_MESH),
        compiler_params=_SC_PARAMS,
        scratch_types=[
            pltpu.VMEM((NPK,), jnp.float32),
            pltpu.VMEM((EW,), jnp.int32),
            pltpu.VMEM((EW,), jnp.int32),
            pltpu.VMEM((EW,), jnp.float32),
            pltpu.VMEM((NP,), jnp.float32),
            pltpu.VMEM((NPK,), jnp.float32),
            pltpu.SemaphoreType.DMA,
        ],
    )
    def agg_norm_kernel(h_hbm, src_hbm, dst_hbm, w_hbm, dinv_hbm,
                        out_hbm, nrm_hbm, hv, srcv, dstv, wv, dv, acc, sem):
        w = _wid()
        sl_edges = pl.ds(w * EW, EW)
        cps = [
            pltpu.async_copy(h_hbm, hv, sem),
            pltpu.async_copy(src_hbm.at[sl_edges], srcv, sem),
            pltpu.async_copy(dst_hbm.at[sl_edges], dstv, sem),
            pltpu.async_copy(w_hbm.at[sl_edges], wv, sem),
            pltpu.async_copy(dinv_hbm, dv, sem),
        ]
        zero = jnp.zeros((L,), jnp.float32)

        @plsc.parallel_loop(0, NPK // L, unroll=8)
        def _(i):
            acc[pl.ds(i * L, L)] = zero

        for cp in cps:
            cp.wait()

        @plsc.parallel_loop(0, steps, unroll=4)
        def _(i):
            sl = pl.ds(i * L, L)
            s16 = srcv[sl]
            d16 = dstv[sl]
            a = plsc.load_gather(dv, [s16])
            b = plsc.load_gather(dv, [d16])
            n16 = a * b * wv[sl]
            wv[sl] = n16
            for k in range(K):
                g = plsc.load_gather(hv, [s16 + (k * NP)])
                plsc.addupdate_scatter(acc, [d16 + (k * NP)], g * n16)

        cp = pltpu.async_copy(wv, nrm_hbm.at[sl_edges], sem)
        pltpu.async_copy(acc, out_hbm.at[w], sem).wait()
        cp.wait()

    return agg_norm_kernel


def _make_agg(ET, NP, K):
    EW = ET // NW
    steps = EW // L
    NPK = NP * K

    @functools.partial(
        pl.kernel,
        out_type=jax.ShapeDtypeStruct((NW, NPK), jnp.float32),
        mesh=plsc.VectorSubcoreMesh(**_MESH),
        compiler_params=_SC_PARAMS,
        scratch_types=[
            pltpu.VMEM((NPK,), jnp.float32),
            pltpu.VMEM((EW,), jnp.int32),
            pltpu.VMEM((EW,), jnp.int32),
            pltpu.VMEM((EW,), jnp.float32),
            pltpu.VMEM((NPK,), jnp.float32),
            pltpu.SemaphoreType.DMA,
        ],
    )
    def agg_kernel(h_hbm, src_hbm, dst_hbm, nrm_hbm, out_hbm,
                   hv, srcv, dstv, nrmv, acc, sem):
        w = _wid()
        sl_edges = pl.ds(w * EW, EW)
        cps = [
            pltpu.async_copy(h_hbm, hv, sem),
            pltpu.async_copy(src_hbm.at[sl_edges], srcv, sem),
            pltpu.async_copy(dst_hbm.at[sl_edges], dstv, sem),
            pltpu.async_copy(nrm_hbm.at[sl_edges], nrmv, sem),
        ]
        zero = jnp.zeros((L,), jnp.float32)

        @plsc.parallel_loop(0, NPK // L, unroll=8)
        def _(i):
            acc[pl.ds(i * L, L)] = zero

        for cp in cps:
            cp.wait()

        @plsc.parallel_loop(0, steps, unroll=4)
        def _(i):
            sl = pl.ds(i * L, L)
            s16 = srcv[sl]
            d16 = dstv[sl]
            n16 = nrmv[sl]
            for k in range(K):
                g = plsc.load_gather(hv, [s16 + (k * NP)])
                plsc.addupdate_scatter(acc, [d16 + (k * NP)], g * n16)

        pltpu.async_copy(acc, out_hbm.at[w], sem).wait()

    return agg_kernel


def _tc_mm(xT, W1T, NP):
    def body(x_ref, w_ref, hp_ref):
        hp_ref[...] = jnp.dot(w_ref[...], x_ref[...],
                              preferred_element_type=jnp.float32)

    return pl.pallas_call(
        body,
        out_shape=jax.ShapeDtypeStruct((W1T.shape[0], NP), jnp.float32),
    )(xT, W1T)


def _tc_dinv(degp, NP):
    def body(degp_ref, dinv_ref):
        deg = lax.max(jnp.sum(degp_ref[...], axis=0), 1.0)
        dinv_ref[...] = lax.rsqrt(deg)

    return pl.pallas_call(
        body,
        out_shape=jax.ShapeDtypeStruct((NP,), jnp.float32),
    )(degp)


def _tc_layer(p, b, WnT, NP, K, last=False, bn=None):
    NW_ = p.shape[0]
    Kn = WnT.shape[0]

    if last:
        def body(p_ref, b_ref, w_ref, bn_ref, o_ref, h_ref):
            i = pl.program_id(0)

            @pl.when(i == 0)
            def _():
                h_ref[...] = p_ref[0]

            @pl.when(i > 0)
            def _():
                h_ref[...] = h_ref[...] + p_ref[0]

            @pl.when(i == NW_ - 1)
            def _():
                h = jnp.tanh(h_ref[...] + b_ref[...])
                h_ref[...] = h
                o_ref[...] = jnp.dot(w_ref[...], h,
                                     preferred_element_type=jnp.float32
                                     ) + bn_ref[...]

        return pl.pallas_call(
            body,
            grid=(NW_,),
            in_specs=[
                pl.BlockSpec((1, K, NP), lambda i: (i, 0, 0)),
                pl.BlockSpec((K, 1), lambda i: (0, 0)),
                pl.BlockSpec((Kn, K), lambda i: (0, 0)),
                pl.BlockSpec((Kn, 1), lambda i: (0, 0)),
            ],
            out_specs=(
                pl.BlockSpec((Kn, NP), lambda i: (0, 0)),
                pl.BlockSpec((K, NP), lambda i: (0, 0)),
            ),
            out_shape=(
                jax.ShapeDtypeStruct((Kn, NP), jnp.float32),
                jax.ShapeDtypeStruct((K, NP), jnp.float32),
            ),
        )(p, b, WnT, bn)

    def body(p_ref, b_ref, w_ref, o_ref, acc_ref):
        i = pl.program_id(0)

        @pl.when(i == 0)
        def _():
            acc_ref[...] = p_ref[0]

        @pl.when(i > 0)
        def _():
            acc_ref[...] = acc_ref[...] + p_ref[0]

        @pl.when(i == NW_ - 1)
        def _():
            h = jnp.tanh(acc_ref[...] + b_ref[...])
            o_ref[...] = jnp.dot(w_ref[...], h,
                                 preferred_element_type=jnp.float32)

    return pl.pallas_call(
        body,
        grid=(NW_,),
        in_specs=[
            pl.BlockSpec((1, K, NP), lambda i: (i, 0, 0)),
            pl.BlockSpec((K, 1), lambda i: (0, 0)),
            pl.BlockSpec((Kn, K), lambda i: (0, 0)),
        ],
        out_specs=pl.BlockSpec((Kn, NP), lambda i: (0, 0)),
        out_shape=jax.ShapeDtypeStruct((Kn, NP), jnp.float32),
        scratch_shapes=[pltpu.VMEM((K, NP), jnp.float32)],
    )(p, b, WnT)


def kernel(x, edge_index, W1, b1, W2, b2, W3, b3, Wc, bc):
    N, D = x.shape
    E = edge_index.shape[1]

    NP = ((N + 511) // 512) * 512
    ET = (((E + N) + 511) // 512) * 512

    xT = jnp.zeros((D, NP), x.dtype).at[:, :N].set(x.T)
    loop = jnp.arange(N, dtype=jnp.int32)
    pad = jnp.zeros((ET - E - N,), jnp.int32)
    srcf = jnp.concatenate([edge_index[0], loop, pad])
    dstf = jnp.concatenate([edge_index[1], loop, pad])
    wf = jnp.concatenate([jnp.ones((E + N,), jnp.float32),
                          jnp.zeros((ET - E - N,), jnp.float32)])

    degp = _make_deg(ET, NP)(dstf, wf)
    hp1 = _tc_mm(xT, W1.T, NP)
    dinv = _tc_dinv(degp, NP)

    K1 = W1.shape[1]
    K2 = W2.shape[1]
    K3 = W3.shape[1]

    p1, nrm = _make_agg_norm(ET, NP, K1)(hp1.reshape(-1), srcf, dstf, wf,
                                         dinv)
    p1 = p1.reshape(NW, K1, NP)
    hp2 = _tc_layer(p1, b1[:, None], W2.T, NP, K1)

    p2 = _make_agg(ET, NP, K2)(hp2.reshape(-1), srcf, dstf,
                               nrm).reshape(NW, K2, NP)
    hp3 = _tc_layer(p2, b2[:, None], W3.T, NP, K2)

    p3 = _make_agg(ET, NP, K3)(hp3.reshape(-1), srcf, dstf,
                               nrm).reshape(NW, K3, NP)
    outT, hT = _tc_layer(p3, b3[:, None], Wc.T, NP, K3, last=True,
                         bn=bc[:, None])

    return (outT.T[:N], hT.T[:N])


# speedup vs baseline: 1.1585x; 1.1585x over previous
import functools

import jax
import jax.numpy as jnp
from jax import lax
from jax.experimental import pallas as pl
from jax.experimental.pallas import tpu as pltpu
from jax.experimental.pallas import tpu_sc as plsc

NC = 2
NS = 16
L = 16
NW = NC * NS

_MESH = dict(core_axis_name="c", subcore_axis_name="s",
             num_cores=NC, num_subcores=NS)
_SC_PARAMS = pltpu.CompilerParams(needs_layout_passes=False)


def _wid():
    return lax.axis_index("c") * NS + lax.axis_index("s")


def _make_agg_norm(ET, NP, K):
    EW = ET // NW
    steps = EW // L
    ET16 = ET // NS
    hsteps = ET16 // L
    CH = NP // NS
    NPK = NP * K
    MAGIC = 0x5F3759DF

    @functools.partial(
        pl.kernel,
        out_type=(jax.ShapeDtypeStruct((NW, NPK), jnp.float32),
                  jax.ShapeDtypeStruct((ET,), jnp.float32),
                  jax.ShapeDtypeStruct((NC, NP), jnp.float32)),
        mesh=plsc.VectorSubcoreMesh(**_MESH),
        compiler_params=_SC_PARAMS,
        scratch_types=[
            pltpu.VMEM((NPK,), jnp.float32),
            pltpu.VMEM((NPK,), jnp.float32),
            pltpu.VMEM((EW,), jnp.int32),
            pltpu.VMEM((EW,), jnp.int32),
            pltpu.VMEM((EW,), jnp.float32),
            pltpu.VMEM((NP,), jnp.float32),
            pltpu.VMEM((CH,), jnp.float32),
            pltpu.SemaphoreType.DMA,
        ],
    )
    def agg_norm_kernel(h_hbm, src_hbm, dst_hbm, dstbits_hbm, w_hbm,
                        out_hbm, nrm_hbm, dinv_hbm,
                        bufh, acc, srcv, dstv, wv, dv, chk, sem):
        c = lax.axis_index("c")
        s = lax.axis_index("s")
        w = c * NS + s
        sl_edges = pl.ds(w * EW, EW)
        sl_hist = pl.ds(s * ET16, ET16)

        cps = [
            pltpu.async_copy(dstbits_hbm.at[sl_hist], acc.at[pl.ds(0, ET16)],
                             sem),
            pltpu.async_copy(w_hbm.at[sl_hist], bufh.at[pl.ds(0, ET16)], sem),
            pltpu.async_copy(src_hbm.at[sl_edges], srcv, sem),
            pltpu.async_copy(dst_hbm.at[sl_edges], dstv, sem),
            pltpu.async_copy(w_hbm.at[sl_edges], wv, sem),
        ]
        zero = jnp.zeros((L,), jnp.float32)

        @plsc.parallel_loop(0, NP // L, unroll=8)
        def _(i):
            dv[pl.ds(i * L, L)] = zero

        for cp in cps:
            cp.wait()

        @plsc.parallel_loop(0, hsteps)
        def _(i):
            sl = pl.ds(i * L, L)
            d16 = plsc.bitcast(acc[sl], jnp.int32)
            plsc.addupdate_scatter(dv, [d16], bufh[sl])

        pltpu.async_copy(dv, out_hbm.at[w, pl.ds(0, NP)], sem).wait()
        plsc.subcore_barrier()

        rps = [
            pltpu.async_copy(
                out_hbm.at[c * NS + j, pl.ds(s * CH, CH)],
                bufh.at[pl.ds(j * CH, CH)], sem)
            for j in range(NS)
        ]
        for cp in rps:
            cp.wait()

        @plsc.parallel_loop(0, CH // L)
        def _(i):
            sl = pl.ds(i * L, L)
            t = bufh[sl]
            for j in range(1, NS):
                t = t + bufh[pl.ds(j * CH + i * L, L)]
            d = jnp.maximum(t, 1.0)
            y = plsc.bitcast(MAGIC - (plsc.bitcast(d, jnp.int32) >> 1),
                             jnp.float32)
            hd = 0.5 * d
            y = y * (1.5 - hd * y * y)
            y = y * (1.5 - hd * y * y)
            y = y * (1.5 - hd * y * y)
            chk[sl] = y

        pltpu.async_copy(chk, dinv_hbm.at[c, pl.ds(s * CH, CH)], sem).wait()
        plsc.subcore_barrier()

        cps = [
            pltpu.async_copy(h_hbm, bufh, sem),
            pltpu.async_copy(dinv_hbm.at[c], dv, sem),
        ]

        @plsc.parallel_loop(0, NPK // L, unroll=8)
        def _(i):
            acc[pl.ds(i * L, L)] = zero

        for cp in cps:
            cp.wait()

        @plsc.parallel_loop(0, steps)
        def _(i):
            sl = pl.ds(i * L, L)
            s16 = srcv[sl]
            d16 = dstv[sl]
            a = plsc.load_gather(dv, [s16])
            b = plsc.load_gather(dv, [d16])
            n16 = a * b * wv[sl]
            wv[sl] = n16
            for k in range(K):
                g = plsc.load_gather(bufh, [s16 + (k * NP)])
                plsc.addupdate_scatter(acc, [d16 + (k * NP)], g * n16)

        cp = pltpu.async_copy(wv, nrm_hbm.at[sl_edges], sem)
        pltpu.async_copy(acc, out_hbm.at[w], sem).wait()
        cp.wait()

    return agg_norm_kernel


def _make_agg(ET, NP, K):
    EW = ET // NW
    steps = EW // L
    NPK = NP * K

    @functools.partial(
        pl.kernel,
        out_type=jax.ShapeDtypeStruct((NW, NPK), jnp.float32),
        mesh=plsc.VectorSubcoreMesh(**_MESH),
        compiler_params=_SC_PARAMS,
        scratch_types=[
            pltpu.VMEM((NPK,), jnp.float32),
            pltpu.VMEM((EW,), jnp.int32),
            pltpu.VMEM((EW,), jnp.int32),
            pltpu.VMEM((EW,), jnp.float32),
            pltpu.VMEM((NPK,), jnp.float32),
            pltpu.SemaphoreType.DMA,
        ],
    )
    def agg_kernel(h_hbm, src_hbm, dst_hbm, nrm_hbm, out_hbm,
                   hv, srcv, dstv, nrmv, acc, sem):
        w = _wid()
        sl_edges = pl.ds(w * EW, EW)
        cps = [
            pltpu.async_copy(h_hbm, hv, sem),
            pltpu.async_copy(src_hbm.at[sl_edges], srcv, sem),
            pltpu.async_copy(dst_hbm.at[sl_edges], dstv, sem),
            pltpu.async_copy(nrm_hbm.at[sl_edges], nrmv, sem),
        ]
        zero = jnp.zeros((L,), jnp.float32)

        @plsc.parallel_loop(0, NPK // L, unroll=8)
        def _(i):
            acc[pl.ds(i * L, L)] = zero

        for cp in cps:
            cp.wait()

        @plsc.parallel_loop(0, steps, unroll=4)
        def _(i):
            sl = pl.ds(i * L, L)
            s16 = srcv[sl]
            d16 = dstv[sl]
            n16 = nrmv[sl]
            for k in range(K):
                g = plsc.load_gather(hv, [s16 + (k * NP)])
                plsc.addupdate_scatter(acc, [d16 + (k * NP)], g * n16)

        pltpu.async_copy(acc, out_hbm.at[w], sem).wait()

    return agg_kernel


def _tc_mm(xT, W1T, NP):
    def body(x_ref, w_ref, hp_ref):
        hp_ref[...] = jnp.dot(w_ref[...], x_ref[...],
                              preferred_element_type=jnp.float32)

    return pl.pallas_call(
        body,
        out_shape=jax.ShapeDtypeStruct((W1T.shape[0], NP), jnp.float32),
    )(xT, W1T)


def _tc_layer(p, b, WnT, NP, K, last=False, bn=None):
    if last:
        def body(p_ref, b_ref, w_ref, bn_ref, o_ref, h_ref):
            agg = jnp.sum(p_ref[...], axis=0) + b_ref[...]
            h = jnp.tanh(agg)
            h_ref[...] = h
            o_ref[...] = jnp.dot(w_ref[...], h,
                                 preferred_element_type=jnp.float32) + bn_ref[...]

        return pl.pallas_call(
            body,
            out_shape=(
                jax.ShapeDtypeStruct((WnT.shape[0], NP), jnp.float32),
                jax.ShapeDtypeStruct((K, NP), jnp.float32),
            ),
        )(p, b, WnT, bn)

    def body(p_ref, b_ref, w_ref, o_ref):
        agg = jnp.sum(p_ref[...], axis=0) + b_ref[...]
        h = jnp.tanh(agg)
        o_ref[...] = jnp.dot(w_ref[...], h,
                             preferred_element_type=jnp.float32)

    return pl.pallas_call(
        body,
        out_shape=jax.ShapeDtypeStruct((WnT.shape[0], NP), jnp.float32),
    )(p, b, WnT)


def kernel(x, edge_index, W1, b1, W2, b2, W3, b3, Wc, bc):
    N, D = x.shape
    E = edge_index.shape[1]

    NP = ((N + 511) // 512) * 512
    ET = (((E + N) + 511) // 512) * 512

    xT = jnp.zeros((D, NP), x.dtype).at[:, :N].set(x.T)
    loop = jnp.arange(N, dtype=jnp.int32)
    pad = jnp.zeros((ET - E - N,), jnp.int32)
    srcf = jnp.concatenate([edge_index[0], loop, pad])
    dstf = jnp.concatenate([edge_index[1], loop, pad])
    wf = jnp.concatenate([jnp.ones((E + N,), jnp.float32),
                          jnp.zeros((ET - E - N,), jnp.float32)])

    hp1 = _tc_mm(xT, W1.T, NP)
    dstbits = lax.bitcast_convert_type(dstf, jnp.float32)

    K1 = W1.shape[1]
    K2 = W2.shape[1]
    K3 = W3.shape[1]

    p1, nrm, _unused_dinv = _make_agg_norm(ET, NP, K1)(
        hp1.reshape(-1), srcf, dstf, dstbits, wf)
    p1 = p1.reshape(NW, K1, NP)
    hp2 = _tc_layer(p1, b1[:, None], W2.T, NP, K1)

    p2 = _make_agg(ET, NP, K2)(hp2.reshape(-1), srcf, dstf,
                               nrm).reshape(NW, K2, NP)
    hp3 = _tc_layer(p2, b2[:, None], W3.T, NP, K2)

    p3 = _make_agg(ET, NP, K3)(hp3.reshape(-1), srcf, dstf,
                               nrm).reshape(NW, K3, NP)
    outT, hT = _tc_layer(p3, b3[:, None], Wc.T, NP, K3, last=True,
                         bn=bc[:, None])

    return (outT.T[:N], hT.T[:N])


# speedup vs baseline: 1.3074x; 1.1285x over previous
import functools

import jax
import jax.numpy as jnp
from jax import lax
from jax.experimental import pallas as pl
from jax.experimental.pallas import tpu as pltpu
from jax.experimental.pallas import tpu_sc as plsc

NC = 2
NS = 16
L = 16
NW = NC * NS

_MESH = dict(core_axis_name="c", subcore_axis_name="s",
             num_cores=NC, num_subcores=NS)
_SC_PARAMS = pltpu.CompilerParams(needs_layout_passes=False)


def _wid():
    return lax.axis_index("c") * NS + lax.axis_index("s")


def _make_deg(ET, NP):
    EW = ET // NW
    steps = EW // L

    @functools.partial(
        pl.kernel,
        out_type=jax.ShapeDtypeStruct((NW, NP), jnp.float32),
        mesh=plsc.VectorSubcoreMesh(**_MESH),
        compiler_params=_SC_PARAMS,
        scratch_types=[
            pltpu.VMEM((EW,), jnp.int32),
            pltpu.VMEM((EW,), jnp.float32),
            pltpu.VMEM((NP,), jnp.float32),
            pltpu.SemaphoreType.DMA,
        ],
    )
    def deg_kernel(sd_hbm, w_hbm, out_hbm, sdv, wv, acc, sem):
        w = _wid()
        sl_edges = pl.ds(w * EW, EW)
        cps = [
            pltpu.async_copy(sd_hbm.at[sl_edges], sdv, sem),
            pltpu.async_copy(w_hbm.at[sl_edges], wv, sem),
        ]
        zero = jnp.zeros((L,), jnp.float32)

        @plsc.parallel_loop(0, NP // L, unroll=8)
        def _(i):
            acc[pl.ds(i * L, L)] = zero

        for cp in cps:
            cp.wait()

        @plsc.parallel_loop(0, steps, unroll=4)
        def _(i):
            sl = pl.ds(i * L, L)
            plsc.addupdate_scatter(acc, [sdv[sl] & 16383], wv[sl])

        pltpu.async_copy(acc, out_hbm.at[w], sem).wait()

    return deg_kernel


def _make_agg_norm(ET, NP, K):
    EW = ET // NW
    steps = EW // L
    NPK = NP * K

    @functools.partial(
        pl.kernel,
        out_type=(jax.ShapeDtypeStruct((NW, NPK), jnp.float32),
                  jax.ShapeDtypeStruct((ET,), jnp.float32)),
        mesh=plsc.VectorSubcoreMesh(**_MESH),
        compiler_params=_SC_PARAMS,
        scratch_types=[
            pltpu.VMEM((NPK,), jnp.float32),
            pltpu.VMEM((EW,), jnp.int32),
            pltpu.VMEM((EW,), jnp.float32),
            pltpu.VMEM((NP,), jnp.float32),
            pltpu.VMEM((NPK,), jnp.float32),
            pltpu.SemaphoreType.DMA,
        ],
    )
    def agg_norm_kernel(h_hbm, sd_hbm, w_hbm, dinv_hbm,
                        out_hbm, nrm_hbm, hv, sdv, wv, dv, acc, sem):
        w = _wid()
        sl_edges = pl.ds(w * EW, EW)
        cps = [
            pltpu.async_copy(h_hbm, hv, sem),
            pltpu.async_copy(sd_hbm.at[sl_edges], sdv, sem),
            pltpu.async_copy(w_hbm.at[sl_edges], wv, sem),
            pltpu.async_copy(dinv_hbm, dv, sem),
        ]
        zero = jnp.zeros((L,), jnp.float32)

        @plsc.parallel_loop(0, NPK // L, unroll=8)
        def _(i):
            acc[pl.ds(i * L, L)] = zero

        for cp in cps:
            cp.wait()

        @plsc.parallel_loop(0, steps, unroll=4)
        def _(i):
            sl = pl.ds(i * L, L)
            sd16 = sdv[sl]
            s16 = lax.shift_right_logical(sd16, 14)
            d16 = sd16 & 16383
            a = plsc.load_gather(dv, [s16])
            b = plsc.load_gather(dv, [d16])
            n16 = a * b * wv[sl]
            wv[sl] = n16
            for k in range(K):
                g = plsc.load_gather(hv, [s16 + (k * NP)])
                plsc.addupdate_scatter(acc, [d16 + (k * NP)], g * n16)

        cp = pltpu.async_copy(wv, nrm_hbm.at[sl_edges], sem)
        pltpu.async_copy(acc, out_hbm.at[w], sem).wait()
        cp.wait()

    return agg_norm_kernel


def _make_agg(ET, NP, K):
    EW = ET // NW
    steps = EW // L
    NPK = NP * K

    @functools.partial(
        pl.kernel,
        out_type=jax.ShapeDtypeStruct((NW, NPK), jnp.float32),
        mesh=plsc.VectorSubcoreMesh(**_MESH),
        compiler_params=_SC_PARAMS,
        scratch_types=[
            pltpu.VMEM((NPK,), jnp.float32),
            pltpu.VMEM((EW,), jnp.int32),
            pltpu.VMEM((EW,), jnp.float32),
            pltpu.VMEM((NPK,), jnp.float32),
            pltpu.SemaphoreType.DMA,
        ],
    )
    def agg_kernel(h_hbm, sd_hbm, nrm_hbm, out_hbm,
                   hv, sdv, nrmv, acc, sem):
        w = _wid()
        sl_edges = pl.ds(w * EW, EW)
        cps = [
            pltpu.async_copy(h_hbm, hv, sem),
            pltpu.async_copy(sd_hbm.at[sl_edges], sdv, sem),
            pltpu.async_copy(nrm_hbm.at[sl_edges], nrmv, sem),
        ]
        zero = jnp.zeros((L,), jnp.float32)

        @plsc.parallel_loop(0, NPK // L, unroll=8)
        def _(i):
            acc[pl.ds(i * L, L)] = zero

        for cp in cps:
            cp.wait()

        @plsc.parallel_loop(0, steps, unroll=4)
        def _(i):
            sl = pl.ds(i * L, L)
            sd16 = sdv[sl]
            s16 = lax.shift_right_logical(sd16, 14)
            d16 = sd16 & 16383
            n16 = nrmv[sl]
            for k in range(K):
                g = plsc.load_gather(hv, [s16 + (k * NP)])
                plsc.addupdate_scatter(acc, [d16 + (k * NP)], g * n16)

        pltpu.async_copy(acc, out_hbm.at[w], sem).wait()

    return agg_kernel


def _tc_prep(xT, W1T, degp, NP):
    def body(x_ref, w_ref, degp_ref, hp_ref, dinv_ref):
        deg = lax.max(jnp.sum(degp_ref[...], axis=0), 1.0)
        dinv_ref[...] = lax.rsqrt(deg)
        hp_ref[...] = jnp.dot(w_ref[...], x_ref[...],
                              preferred_element_type=jnp.float32)

    return pl.pallas_call(
        body,
        out_shape=(
            jax.ShapeDtypeStruct((W1T.shape[0], NP), jnp.float32),
            jax.ShapeDtypeStruct((NP,), jnp.float32),
        ),
    )(xT, W1T, degp)


def _tc_layer(p, b, WnT, NP, K, last=False, bn=None):
    if last:
        def body(p_ref, b_ref, w_ref, bn_ref, o_ref, h_ref):
            agg = jnp.sum(p_ref[...], axis=0) + b_ref[...]
            h = jnp.tanh(agg)
            h_ref[...] = h
            o_ref[...] = jnp.dot(w_ref[...], h,
                                 preferred_element_type=jnp.float32) + bn_ref[...]

        return pl.pallas_call(
            body,
            out_shape=(
                jax.ShapeDtypeStruct((WnT.shape[0], NP), jnp.float32),
                jax.ShapeDtypeStruct((K, NP), jnp.float32),
            ),
        )(p, b, WnT, bn)

    def body(p_ref, b_ref, w_ref, o_ref):
        agg = jnp.sum(p_ref[...], axis=0) + b_ref[...]
        h = jnp.tanh(agg)
        o_ref[...] = jnp.dot(w_ref[...], h,
                             preferred_element_type=jnp.float32)

    return pl.pallas_call(
        body,
        out_shape=jax.ShapeDtypeStruct((WnT.shape[0], NP), jnp.float32),
    )(p, b, WnT)


def kernel(x, edge_index, W1, b1, W2, b2, W3, b3, Wc, bc):
    N, D = x.shape
    E = edge_index.shape[1]

    NP = ((N + 511) // 512) * 512
    ET = (((E + N) + 511) // 512) * 512

    xT = jnp.zeros((D, NP), x.dtype).at[:, :N].set(x.T)
    loop = jnp.arange(N, dtype=jnp.int32)
    pad = jnp.zeros((ET - E - N,), jnp.int32)
    srcf = jnp.concatenate([edge_index[0], loop, pad])
    dstf = jnp.concatenate([edge_index[1], loop, pad])
    sdf = srcf * 16384 + dstf
    wf = jnp.concatenate([jnp.ones((E + N,), jnp.float32),
                          jnp.zeros((ET - E - N,), jnp.float32)])

    degp = _make_deg(ET, NP)(sdf, wf)
    hp1, dinv = _tc_prep(xT, W1.T, degp, NP)

    K1 = W1.shape[1]
    K2 = W2.shape[1]
    K3 = W3.shape[1]

    p1, nrm = _make_agg_norm(ET, NP, K1)(hp1.reshape(-1), sdf, wf, dinv)
    p1 = p1.reshape(NW, K1, NP)
    hp2 = _tc_layer(p1, b1[:, None], W2.T, NP, K1)

    p2 = _make_agg(ET, NP, K2)(hp2.reshape(-1), sdf,
                               nrm).reshape(NW, K2, NP)
    hp3 = _tc_layer(p2, b2[:, None], W3.T, NP, K2)

    p3 = _make_agg(ET, NP, K3)(hp3.reshape(-1), sdf,
                               nrm).reshape(NW, K3, NP)
    outT, hT = _tc_layer(p3, b3[:, None], Wc.T, NP, K3, last=True,
                         bn=bc[:, None])

    return (outT.T[:N], hT.T[:N])


# speedup vs baseline: 1.3285x; 1.0162x over previous
import functools

import jax
import jax.numpy as jnp
from jax import lax
from jax.experimental import pallas as pl
from jax.experimental.pallas import tpu as pltpu
from jax.experimental.pallas import tpu_sc as plsc

NC = 2
NS = 16
L = 16
NW = NC * NS

_MESH = dict(core_axis_name="c", subcore_axis_name="s",
             num_cores=NC, num_subcores=NS)
_SC_PARAMS = pltpu.CompilerParams(needs_layout_passes=False)


def _wid():
    return lax.axis_index("c") * NS + lax.axis_index("s")


def _make_deg(ET, NP):
    EW = ET // NW
    steps = EW // L

    @functools.partial(
        pl.kernel,
        out_type=jax.ShapeDtypeStruct((NW, NP), jnp.float32),
        mesh=plsc.VectorSubcoreMesh(**_MESH),
        compiler_params=_SC_PARAMS,
        scratch_types=[
            pltpu.VMEM((EW,), jnp.int32),
            pltpu.VMEM((NP,), jnp.float32),
            pltpu.SemaphoreType.DMA,
        ],
    )
    def deg_kernel(sd_hbm, out_hbm, sdv, acc, sem):
        w = _wid()
        sl_edges = pl.ds(w * EW, EW)
        one = jnp.ones((L,), jnp.float32)
        cps = [
            pltpu.async_copy(sd_hbm.at[sl_edges], sdv, sem),
        ]
        zero = jnp.zeros((L,), jnp.float32)

        @plsc.parallel_loop(0, NP // L, unroll=8)
        def _(i):
            acc[pl.ds(i * L, L)] = zero

        for cp in cps:
            cp.wait()

        @plsc.parallel_loop(0, steps, unroll=4)
        def _(i):
            sl = pl.ds(i * L, L)
            plsc.addupdate_scatter(acc, [sdv[sl] & 16383], one)

        pltpu.async_copy(acc, out_hbm.at[w], sem).wait()

    return deg_kernel


def _make_agg_norm(ET, NP, K):
    EW = ET // NW
    steps = EW // L
    NPK = NP * K

    @functools.partial(
        pl.kernel,
        out_type=(jax.ShapeDtypeStruct((NW, NPK), jnp.float32),
                  jax.ShapeDtypeStruct((ET,), jnp.float32)),
        mesh=plsc.VectorSubcoreMesh(**_MESH),
        compiler_params=_SC_PARAMS,
        scratch_types=[
            pltpu.VMEM((NPK,), jnp.float32),
            pltpu.VMEM((EW,), jnp.int32),
            pltpu.VMEM((EW,), jnp.float32),
            pltpu.VMEM((NP,), jnp.float32),
            pltpu.VMEM((NPK,), jnp.float32),
            pltpu.SemaphoreType.DMA,
        ],
    )
    def agg_norm_kernel(h_hbm, sd_hbm, dinv_hbm,
                        out_hbm, nrm_hbm, hv, sdv, wv, dv, acc, sem):
        w = _wid()
        sl_edges = pl.ds(w * EW, EW)
        cps = [
            pltpu.async_copy(h_hbm, hv, sem),
            pltpu.async_copy(sd_hbm.at[sl_edges], sdv, sem),
            pltpu.async_copy(dinv_hbm, dv, sem),
        ]
        zero = jnp.zeros((L,), jnp.float32)

        @plsc.parallel_loop(0, NPK // L, unroll=8)
        def _(i):
            acc[pl.ds(i * L, L)] = zero

        for cp in cps:
            cp.wait()

        @plsc.parallel_loop(0, steps, unroll=4)
        def _(i):
            sl = pl.ds(i * L, L)
            sd16 = sdv[sl]
            s16 = lax.shift_right_logical(sd16, 14)
            d16 = sd16 & 16383
            a = plsc.load_gather(dv, [s16])
            b = plsc.load_gather(dv, [d16])
            n16 = a * b
            wv[sl] = n16
            for k in range(K):
                g = plsc.load_gather(hv, [s16 + (k * NP)])
                plsc.addupdate_scatter(acc, [d16 + (k * NP)], g * n16)

        cp = pltpu.async_copy(wv, nrm_hbm.at[sl_edges], sem)
        pltpu.async_copy(acc, out_hbm.at[w], sem).wait()
        cp.wait()

    return agg_norm_kernel


def _make_agg(ET, NP, K):
    EW = ET // NW
    steps = EW // L
    NPK = NP * K

    @functools.partial(
        pl.kernel,
        out_type=jax.ShapeDtypeStruct((NW, NPK), jnp.float32),
        mesh=plsc.VectorSubcoreMesh(**_MESH),
        compiler_params=_SC_PARAMS,
        scratch_types=[
            pltpu.VMEM((NPK,), jnp.float32),
            pltpu.VMEM((EW,), jnp.int32),
            pltpu.VMEM((EW,), jnp.float32),
            pltpu.VMEM((NPK,), jnp.float32),
            pltpu.SemaphoreType.DMA,
        ],
    )
    def agg_kernel(h_hbm, sd_hbm, nrm_hbm, out_hbm,
                   hv, sdv, nrmv, acc, sem):
        w = _wid()
        sl_edges = pl.ds(w * EW, EW)
        cps = [
            pltpu.async_copy(h_hbm, hv, sem),
            pltpu.async_copy(sd_hbm.at[sl_edges], sdv, sem),
            pltpu.async_copy(nrm_hbm.at[sl_edges], nrmv, sem),
        ]
        zero = jnp.zeros((L,), jnp.float32)

        @plsc.parallel_loop(0, NPK // L, unroll=8)
        def _(i):
            acc[pl.ds(i * L, L)] = zero

        for cp in cps:
            cp.wait()

        @plsc.parallel_loop(0, steps, unroll=4)
        def _(i):
            sl = pl.ds(i * L, L)
            sd16 = sdv[sl]
            s16 = lax.shift_right_logical(sd16, 14)
            d16 = sd16 & 16383
            n16 = nrmv[sl]
            for k in range(K):
                g = plsc.load_gather(hv, [s16 + (k * NP)])
                plsc.addupdate_scatter(acc, [d16 + (k * NP)], g * n16)

        pltpu.async_copy(acc, out_hbm.at[w], sem).wait()

    return agg_kernel


def _tc_prep(xT, W1T, degp, NP):
    def body(x_ref, w_ref, degp_ref, hp_ref, dinv_ref):
        deg = lax.max(jnp.sum(degp_ref[...], axis=0), 1.0)
        dinv_ref[...] = lax.rsqrt(deg)
        hp_ref[...] = jnp.dot(w_ref[...], x_ref[...],
                              preferred_element_type=jnp.float32)

    return pl.pallas_call(
        body,
        out_shape=(
            jax.ShapeDtypeStruct((W1T.shape[0], NP), jnp.float32),
            jax.ShapeDtypeStruct((NP,), jnp.float32),
        ),
    )(xT, W1T, degp)


def _tc_layer(p, b, WnT, NP, K, last=False, bn=None):
    if last:
        def body(p_ref, b_ref, w_ref, bn_ref, o_ref, h_ref):
            agg = jnp.sum(p_ref[...], axis=0) + b_ref[...]
            h = jnp.tanh(agg)
            h_ref[...] = h
            o_ref[...] = jnp.dot(w_ref[...], h,
                                 preferred_element_type=jnp.float32) + bn_ref[...]

        return pl.pallas_call(
            body,
            out_shape=(
                jax.ShapeDtypeStruct((WnT.shape[0], NP), jnp.float32),
                jax.ShapeDtypeStruct((K, NP), jnp.float32),
            ),
        )(p, b, WnT, bn)

    def body(p_ref, b_ref, w_ref, o_ref):
        agg = jnp.sum(p_ref[...], axis=0) + b_ref[...]
        h = jnp.tanh(agg)
        o_ref[...] = jnp.dot(w_ref[...], h,
                             preferred_element_type=jnp.float32)

    return pl.pallas_call(
        body,
        out_shape=jax.ShapeDtypeStruct((WnT.shape[0], NP), jnp.float32),
    )(p, b, WnT)


def kernel(x, edge_index, W1, b1, W2, b2, W3, b3, Wc, bc):
    N, D = x.shape
    E = edge_index.shape[1]

    NP = ((N + 511) // 512) * 512
    ET = (((E + N) + 511) // 512) * 512

    xT = jnp.zeros((D, NP), x.dtype).at[:, :N].set(x.T)
    loop = jnp.arange(N, dtype=jnp.int32)
    pad = jnp.full((ET - E - N,), N, jnp.int32)
    srcf = jnp.concatenate([edge_index[0], loop, pad])
    dstf = jnp.concatenate([edge_index[1], loop, pad])
    sdf = srcf * 16384 + dstf

    degp = _make_deg(ET, NP)(sdf)
    hp1, dinv = _tc_prep(xT, W1.T, degp, NP)

    K1 = W1.shape[1]
    K2 = W2.shape[1]
    K3 = W3.shape[1]

    p1, nrm = _make_agg_norm(ET, NP, K1)(hp1.reshape(-1), sdf, dinv)
    p1 = p1.reshape(NW, K1, NP)
    hp2 = _tc_layer(p1, b1[:, None], W2.T, NP, K1)

    p2 = _make_agg(ET, NP, K2)(hp2.reshape(-1), sdf,
                               nrm).reshape(NW, K2, NP)
    hp3 = _tc_layer(p2, b2[:, None], W3.T, NP, K2)

    p3 = _make_agg(ET, NP, K3)(hp3.reshape(-1), sdf,
                               nrm).reshape(NW, K3, NP)
    outT, hT = _tc_layer(p3, b3[:, None], Wc.T, NP, K3, last=True,
                         bn=bc[:, None])

    return (outT.T[:N], hT.T[:N])
